# baseline (speedup 1.0000x reference)
import functools

import jax
import jax.numpy as jnp
import numpy as np
from jax import lax
from jax.experimental import pallas as pl
from jax.experimental.pallas import tpu as pltpu
from jax.experimental.pallas import tpu_sc as plsc

B, P, Cv = 4, 576, 768
V, Dnb, K, H = 100000, 300, 3, 768
R = B * P
RB = 2304
NRB = R // RB
VB = 512
NVB = (V + VB - 1) // VB
DP = 384
INT_MAX = np.int32(2**31 - 1)
NEG = np.float32(-np.inf)


def _simtopk_body(vseq_ref, t_ref, vals_ref, idx_ref, qbf_scr, sa_scr,
                  sb_scr, io_scr, rv_scr, ri_scr):
    i = pl.program_id(0)
    j = pl.program_id(1)

    @pl.when(jnp.logical_and(i == 0, j == 0))
    def _once():
        io_scr[...] = lax.broadcasted_iota(jnp.int32, (RB, VB), 1)

    @pl.when(j == 0)
    def _init():
        v = vseq_ref[pl.ds(i * RB, RB), :]
        nrm = jnp.sqrt(jnp.sum(v * v, axis=1, keepdims=True))
        q = v / jnp.maximum(nrm, 1e-12)
        qbf_scr[...] = q.astype(jnp.bfloat16)
        rv_scr[...] = jnp.full((RB, 8), NEG, jnp.float32)
        ri_scr[...] = jnp.zeros((RB, 8), jnp.int32)

    q = qbf_scr[...]
    t = t_ref[...].astype(jnp.bfloat16)

    def _update(s_raw):
        jj = j - 1
        ids = io_scr[...]
        bound = jnp.where(j == 0, 0, V - jj * VB)
        s = jnp.where(ids < bound, s_raw, NEG)

        bv, bi = [], []
        masked = s
        for m in range(K):
            mx = jnp.max(masked, axis=1, keepdims=True)
            am = jnp.min(jnp.where(masked == mx, ids, INT_MAX), axis=1,
                         keepdims=True)
            bv.append(mx)
            bi.append(am + jj * VB)
            if m < K - 1:
                masked = jnp.where(ids == am, NEG, masked)

        cv = jnp.concatenate([rv_scr[...][:, :K]] + bv, axis=1)
        ci = jnp.concatenate([ri_scr[...][:, :K]] + bi, axis=1)
        mv, mi = [], []
        for m in range(K):
            mx = jnp.max(cv, axis=1, keepdims=True)
            am = jnp.min(jnp.where(cv == mx, ci, INT_MAX), axis=1,
                         keepdims=True)
            mv.append(mx)
            mi.append(am)
            if m < K - 1:
                cv = jnp.where(ci == am, NEG, cv)
        rv_scr[...] = jnp.concatenate(
            mv + [jnp.full((RB, 8 - K), NEG, jnp.float32)], axis=1)
        ri_scr[...] = jnp.concatenate(
            mi + [jnp.zeros((RB, 8 - K), jnp.int32)], axis=1)

    dims = (((1,), (1,)), ((), ()))

    @pl.when(j % 2 == 0)
    def _even():
        sa_scr[...] = lax.dot_general(q, t, dims,
                                      preferred_element_type=jnp.float32)
        _update(sb_scr[...])

    @pl.when(j % 2 == 1)
    def _odd():
        sb_scr[...] = lax.dot_general(q, t, dims,
                                      preferred_element_type=jnp.float32)
        _update(sa_scr[...])

    @pl.when(j == NVB)
    def _emit():
        nv = rv_scr[...]
        ni = ri_scr[...]
        vals_ref[pl.ds(i * RB, RB), :] = jnp.concatenate(
            [nv[:, :K], jnp.zeros((RB, 128 - K), jnp.float32)], axis=1)
        idx_ref[pl.ds(i * RB, RB), :] = ni[:, :K]


def _sim_topk(v_seq2d, t_clip):
    grid = (NRB, NVB + 1)
    return pl.pallas_call(
        _simtopk_body,
        grid=grid,
        in_specs=[
            pl.BlockSpec((R, Cv), lambda i, j: (0, 0)),
            pl.BlockSpec((VB, Cv), lambda i, j: (jnp.minimum(j, NVB - 1), 0)),
        ],
        out_specs=[
            pl.BlockSpec((R, 128), lambda i, j: (0, 0)),
            pl.BlockSpec((R, K), lambda i, j: (0, 0)),
        ],
        out_shape=[
            jax.ShapeDtypeStruct((R, 128), jnp.float32),
            jax.ShapeDtypeStruct((R, K), jnp.int32),
        ],
        scratch_shapes=[
            pltpu.VMEM((RB, Cv), jnp.bfloat16),
            pltpu.VMEM((RB, VB), jnp.float32),
            pltpu.VMEM((RB, VB), jnp.float32),
            pltpu.VMEM((RB, VB), jnp.int32),
            pltpu.VMEM((RB, 8), jnp.float32),
            pltpu.VMEM((RB, 8), jnp.int32),
        ],
        compiler_params=pltpu.CompilerParams(
            dimension_semantics=("arbitrary", "arbitrary"),
        ),
    )(v_seq2d, t_clip)


PCH = 160
PSPAN = 3200


def _pad_table_sc(nb_vecs):
    mesh = plsc.VectorSubcoreMesh(core_axis_name="c", subcore_axis_name="s")

    @functools.partial(
        pl.kernel,
        mesh=mesh,
        out_type=jax.ShapeDtypeStruct((V, DP), jnp.float32),
        scratch_types=[
            pltpu.VMEM((PCH, Dnb), jnp.float32),
            pltpu.VMEM((PCH, DP), jnp.float32),
            pltpu.SemaphoreType.DMA,
        ],
    )
    def k(nb_hbm, out_hbm, bufin, buf, sem):
        wid = lax.axis_index("s") * 2 + lax.axis_index("c")
        base = wid * PSPAN
        zeros16 = jnp.zeros((16,), jnp.float32)

        def zrow(r, _):
            for c in (Dnb, Dnb + 16, Dnb + 32, Dnb + 48, Dnb + 64, DP - 16):
                buf[r, pl.ds(c, 16)] = zeros16
            return 0
        lax.fori_loop(0, PCH, zrow, 0)

        cols = tuple(range(0, Dnb - 16, 16)) + (Dnb - 16,)

        def chunk(n, _):
            r0 = base + n * PCH
            pltpu.async_copy(nb_hbm.at[pl.ds(r0, PCH), :], bufin, sem).wait()

            def arow(r, _):
                for c in cols:
                    buf[r, pl.ds(c, 16)] = bufin[r, pl.ds(c, 16)]
                return 0
            lax.fori_loop(0, PCH, arow, 0)
            pltpu.async_copy(buf, out_hbm.at[pl.ds(r0, PCH), :], sem).wait()
            return 0

        nchunks = jnp.where(wid == 31, (V - 31 * PSPAN) // PCH, PSPAN // PCH)
        lax.fori_loop(0, nchunks, chunk, 0)

    return k(nb_vecs)


NW = 32
BPW = (R * K) // NW


def _gather_sc(nb_pad, idx_flat):
    mesh = plsc.VectorSubcoreMesh(core_axis_name="c", subcore_axis_name="s")

    @functools.partial(
        pl.kernel,
        mesh=mesh,
        out_type=jax.ShapeDtypeStruct((R * K, DP), jnp.float32),
        scratch_types=[
            pltpu.VMEM((BPW,), jnp.int32),
            pltpu.VMEM((BPW, DP), jnp.float32),
            pltpu.SemaphoreType.DMA,
        ],
    )
    def k(table_hbm, idx_hbm, out_hbm, idx_v, rows_v, sem):
        wid = lax.axis_index("s") * 2 + lax.axis_index("c")
        base = wid * BPW
        pltpu.sync_copy(idx_hbm.at[pl.ds(base, BPW)], idx_v)
        pltpu.async_copy(table_hbm.at[idx_v], rows_v, sem).wait()
        pltpu.sync_copy(rows_v, out_hbm.at[pl.ds(base, BPW)])

    return k(nb_pad, idx_flat)


MRB = 576


def _mlp_body(vseq_ref, nbv_ref, valp_ref, w1a_ref, w1b_ref, w1c_ref,
              b1_ref, w2_ref, b2_ref, out_ref):
    dims = (((1,), (0,)), ((), ()))
    acc = lax.dot_general(vseq_ref[...].astype(jnp.bfloat16),
                          w1a_ref[...].astype(jnp.bfloat16), dims,
                          preferred_element_type=jnp.float32)
    acc += lax.dot_general(nbv_ref[...].astype(jnp.bfloat16),
                           w1b_ref[...].astype(jnp.bfloat16), dims,
                           preferred_element_type=jnp.float32)
    acc += lax.dot_general(valp_ref[...].astype(jnp.bfloat16),
                           w1c_ref[...].astype(jnp.bfloat16), dims,
                           preferred_element_type=jnp.float32)
    acc += b1_ref[...]
    h = 0.5 * acc * (1.0 + lax.erf(acc * np.float32(1.0 / np.sqrt(2.0))))
    f = lax.dot_general(h.astype(jnp.bfloat16),
                        w2_ref[...].astype(jnp.bfloat16), dims,
                        preferred_element_type=jnp.float32)
    f += b2_ref[...]
    mu = jnp.mean(f, axis=1, keepdims=True)
    c = f - mu
    var = jnp.mean(c * c, axis=1, keepdims=True)
    out_ref[...] = c * lax.rsqrt(var + 1e-5)


def _mlp(v_seq2d, nbv, valp, w1a, w1b, w1c, b1, w2, b2):
    grid = (R // MRB,)
    return pl.pallas_call(
        _mlp_body,
        grid=grid,
        in_specs=[
            pl.BlockSpec((MRB, Cv), lambda i: (i, 0)),
            pl.BlockSpec((MRB, K * DP), lambda i: (i, 0)),
            pl.BlockSpec((MRB, 128), lambda i: (i, 0)),
            pl.BlockSpec((Cv, H), lambda i: (0, 0)),
            pl.BlockSpec((K * DP, H), lambda i: (0, 0)),
            pl.BlockSpec((128, H), lambda i: (0, 0)),
            pl.BlockSpec((1, H), lambda i: (0, 0)),
            pl.BlockSpec((H, Cv), lambda i: (0, 0)),
            pl.BlockSpec((1, Cv), lambda i: (0, 0)),
        ],
        out_specs=pl.BlockSpec((MRB, Cv), lambda i: (i, 0)),
        out_shape=jax.ShapeDtypeStruct((R, Cv), jnp.float32),
        compiler_params=pltpu.CompilerParams(
            dimension_semantics=("arbitrary",),
        ),
    )(v_seq2d, nbv, valp, w1a, w1b, w1c, b1, w2, b2)


def kernel(v_seq, T_clip, nb_vecs, W1, b1, W2, b2):
    v2d = v_seq.reshape(R, Cv)
    valp, idx = _sim_topk(v2d, T_clip)

    nb_pad = jnp.concatenate(
        [nb_vecs, jnp.zeros((V, DP - Dnb), jnp.float32)], axis=1)
    rows = _gather_sc(nb_pad, idx.reshape(R * K))
    nbv = rows.reshape(R, K * DP)

    w1a = W1[:Cv]
    w1b = jnp.concatenate(
        [W1[Cv:Cv + K * Dnb].reshape(K, Dnb, H),
         jnp.zeros((K, DP - Dnb, H), jnp.float32)], axis=1).reshape(K * DP, H)
    w1c = jnp.concatenate(
        [W1[Cv + K * Dnb:], jnp.zeros((128 - K, H), jnp.float32)], axis=0)

    out = _mlp(v2d, nbv, valp, w1a, w1b, w1c,
               b1.reshape(1, H), W2, b2.reshape(1, Cv))
    return out.reshape(B, P, Cv)


# speedup vs baseline: 1.4378x; 1.4378x over previous
import functools

import jax
import jax.numpy as jnp
import numpy as np
from jax import lax
from jax.experimental import pallas as pl
from jax.experimental.pallas import tpu as pltpu
from jax.experimental.pallas import tpu_sc as plsc

B, P, Cv = 4, 576, 768
V, Dnb, K, H = 100000, 300, 3, 768
R = B * P
RB = 576
NRB = R // RB
VB = 2048
NVB = (V + VB - 1) // VB
DP = 384
INT_MAX = np.int32(2**31 - 1)
NEG = np.float32(-np.inf)


def _simtopk_body(vseq_ref, t_ref, vals_ref, idx_ref, qbf_scr, sa_scr,
                  sb_scr, io_scr, rv_scr, ri_scr):
    i = pl.program_id(0)
    j = pl.program_id(1)

    @pl.when(jnp.logical_and(i == 0, j == 0))
    def _once():
        io_scr[...] = lax.broadcasted_iota(jnp.int32, (RB, VB), 1)

    @pl.when(j == 0)
    def _init():
        v = vseq_ref[pl.ds(i * RB, RB), :]
        nrm = jnp.sqrt(jnp.sum(v * v, axis=1, keepdims=True))
        q = v / jnp.maximum(nrm, 1e-12)
        qbf_scr[...] = q.astype(jnp.bfloat16)
        rv_scr[...] = jnp.full((RB, 8), NEG, jnp.float32)
        ri_scr[...] = jnp.zeros((RB, 8), jnp.int32)

    q = qbf_scr[...]
    t = t_ref[...].astype(jnp.bfloat16)

    def _update(s_raw):
        jj = j - 1
        ids = io_scr[...]
        bound = jnp.where(j == 0, 0, V - jj * VB)
        s = jnp.where(ids < bound, s_raw, NEG)

        bv, bi = [], []
        masked = s
        for m in range(K):
            mx = jnp.max(masked, axis=1, keepdims=True)
            am = jnp.min(jnp.where(masked == mx, ids, INT_MAX), axis=1,
                         keepdims=True)
            bv.append(mx)
            bi.append(am + jj * VB)
            if m < K - 1:
                masked = jnp.where(ids == am, NEG, masked)

        cv = jnp.concatenate([rv_scr[...][:, :K]] + bv, axis=1)
        ci = jnp.concatenate([ri_scr[...][:, :K]] + bi, axis=1)
        mv, mi = [], []
        for m in range(K):
            mx = jnp.max(cv, axis=1, keepdims=True)
            am = jnp.min(jnp.where(cv == mx, ci, INT_MAX), axis=1,
                         keepdims=True)
            mv.append(mx)
            mi.append(am)
            if m < K - 1:
                cv = jnp.where(ci == am, NEG, cv)
        rv_scr[...] = jnp.concatenate(
            mv + [jnp.full((RB, 8 - K), NEG, jnp.float32)], axis=1)
        ri_scr[...] = jnp.concatenate(
            mi + [jnp.zeros((RB, 8 - K), jnp.int32)], axis=1)

    dims = (((1,), (1,)), ((), ()))

    @pl.when(j % 2 == 0)
    def _even():
        sa_scr[...] = lax.dot_general(q, t, dims,
                                      preferred_element_type=jnp.float32)
        _update(sb_scr[...])

    @pl.when(j % 2 == 1)
    def _odd():
        sb_scr[...] = lax.dot_general(q, t, dims,
                                      preferred_element_type=jnp.float32)
        _update(sa_scr[...])

    @pl.when(j == NVB)
    def _emit():
        nv = rv_scr[...]
        ni = ri_scr[...]
        vals_ref[pl.ds(i * RB, RB), :] = jnp.concatenate(
            [nv[:, :K], jnp.zeros((RB, 128 - K), jnp.float32)], axis=1)
        idx_ref[pl.ds(i * RB, RB), :] = ni[:, :K]


def _sim_topk(v_seq2d, t_clip):
    grid = (NRB, NVB + 1)
    return pl.pallas_call(
        _simtopk_body,
        grid=grid,
        in_specs=[
            pl.BlockSpec((R, Cv), lambda i, j: (0, 0)),
            pl.BlockSpec((VB, Cv), lambda i, j: (jnp.minimum(j, NVB - 1), 0)),
        ],
        out_specs=[
            pl.BlockSpec((R, 128), lambda i, j: (0, 0)),
            pl.BlockSpec((R, K), lambda i, j: (0, 0)),
        ],
        out_shape=[
            jax.ShapeDtypeStruct((R, 128), jnp.float32),
            jax.ShapeDtypeStruct((R, K), jnp.int32),
        ],
        scratch_shapes=[
            pltpu.VMEM((RB, Cv), jnp.bfloat16),
            pltpu.VMEM((RB, VB), jnp.float32),
            pltpu.VMEM((RB, VB), jnp.float32),
            pltpu.VMEM((RB, VB), jnp.int32),
            pltpu.VMEM((RB, 8), jnp.float32),
            pltpu.VMEM((RB, 8), jnp.int32),
        ],
        compiler_params=pltpu.CompilerParams(
            dimension_semantics=("arbitrary", "arbitrary"),
        ),
    )(v_seq2d, t_clip)


PCH = 160
PSPAN = 3200


def _pad_table_sc(nb_vecs):
    mesh = plsc.VectorSubcoreMesh(core_axis_name="c", subcore_axis_name="s")

    @functools.partial(
        pl.kernel,
        mesh=mesh,
        out_type=jax.ShapeDtypeStruct((V, DP), jnp.float32),
        scratch_types=[
            pltpu.VMEM((PCH, Dnb), jnp.float32),
            pltpu.VMEM((PCH, DP), jnp.float32),
            pltpu.SemaphoreType.DMA,
        ],
    )
    def k(nb_hbm, out_hbm, bufin, buf, sem):
        wid = lax.axis_index("s") * 2 + lax.axis_index("c")
        base = wid * PSPAN
        zeros16 = jnp.zeros((16,), jnp.float32)

        def zrow(r, _):
            for c in (Dnb, Dnb + 16, Dnb + 32, Dnb + 48, Dnb + 64, DP - 16):
                buf[r, pl.ds(c, 16)] = zeros16
            return 0
        lax.fori_loop(0, PCH, zrow, 0)

        cols = tuple(range(0, Dnb - 16, 16)) + (Dnb - 16,)

        def chunk(n, _):
            r0 = base + n * PCH
            pltpu.async_copy(nb_hbm.at[pl.ds(r0, PCH), :], bufin, sem).wait()

            def arow(r, _):
                for c in cols:
                    buf[r, pl.ds(c, 16)] = bufin[r, pl.ds(c, 16)]
                return 0
            lax.fori_loop(0, PCH, arow, 0)
            pltpu.async_copy(buf, out_hbm.at[pl.ds(r0, PCH), :], sem).wait()
            return 0

        nchunks = jnp.where(wid == 31, (V - 31 * PSPAN) // PCH, PSPAN // PCH)
        lax.fori_loop(0, nchunks, chunk, 0)

    return k(nb_vecs)


NW = 32
BPW = (R * K) // NW


def _gather_sc(nb_pad, idx_flat):
    mesh = plsc.VectorSubcoreMesh(core_axis_name="c", subcore_axis_name="s")

    @functools.partial(
        pl.kernel,
        mesh=mesh,
        out_type=jax.ShapeDtypeStruct((R * K, DP), jnp.float32),
        scratch_types=[
            pltpu.VMEM((BPW,), jnp.int32),
            pltpu.VMEM((BPW, DP), jnp.float32),
            pltpu.SemaphoreType.DMA,
        ],
    )
    def k(table_hbm, idx_hbm, out_hbm, idx_v, rows_v, sem):
        wid = lax.axis_index("s") * 2 + lax.axis_index("c")
        base = wid * BPW
        pltpu.sync_copy(idx_hbm.at[pl.ds(base, BPW)], idx_v)
        pltpu.async_copy(table_hbm.at[idx_v], rows_v, sem).wait()
        pltpu.sync_copy(rows_v, out_hbm.at[pl.ds(base, BPW)])

    return k(nb_pad, idx_flat)


MRB = 576


def _mlp_body(vseq_ref, nbv_ref, valp_ref, w1a_ref, w1b_ref, w1c_ref,
              b1_ref, w2_ref, b2_ref, out_ref):
    dims = (((1,), (0,)), ((), ()))
    acc = lax.dot_general(vseq_ref[...].astype(jnp.bfloat16),
                          w1a_ref[...].astype(jnp.bfloat16), dims,
                          preferred_element_type=jnp.float32)
    acc += lax.dot_general(nbv_ref[...].astype(jnp.bfloat16),
                           w1b_ref[...].astype(jnp.bfloat16), dims,
                           preferred_element_type=jnp.float32)
    acc += lax.dot_general(valp_ref[...].astype(jnp.bfloat16),
                           w1c_ref[...].astype(jnp.bfloat16), dims,
                           preferred_element_type=jnp.float32)
    acc += b1_ref[...]
    h = 0.5 * acc * (1.0 + lax.erf(acc * np.float32(1.0 / np.sqrt(2.0))))
    f = lax.dot_general(h.astype(jnp.bfloat16),
                        w2_ref[...].astype(jnp.bfloat16), dims,
                        preferred_element_type=jnp.float32)
    f += b2_ref[...]
    mu = jnp.mean(f, axis=1, keepdims=True)
    c = f - mu
    var = jnp.mean(c * c, axis=1, keepdims=True)
    out_ref[...] = c * lax.rsqrt(var + 1e-5)


def _mlp(v_seq2d, nbv, valp, w1a, w1b, w1c, b1, w2, b2):
    grid = (R // MRB,)
    return pl.pallas_call(
        _mlp_body,
        grid=grid,
        in_specs=[
            pl.BlockSpec((MRB, Cv), lambda i: (i, 0)),
            pl.BlockSpec((MRB, K * DP), lambda i: (i, 0)),
            pl.BlockSpec((MRB, 128), lambda i: (i, 0)),
            pl.BlockSpec((Cv, H), lambda i: (0, 0)),
            pl.BlockSpec((K * DP, H), lambda i: (0, 0)),
            pl.BlockSpec((128, H), lambda i: (0, 0)),
            pl.BlockSpec((1, H), lambda i: (0, 0)),
            pl.BlockSpec((H, Cv), lambda i: (0, 0)),
            pl.BlockSpec((1, Cv), lambda i: (0, 0)),
        ],
        out_specs=pl.BlockSpec((MRB, Cv), lambda i: (i, 0)),
        out_shape=jax.ShapeDtypeStruct((R, Cv), jnp.float32),
        compiler_params=pltpu.CompilerParams(
            dimension_semantics=("arbitrary",),
        ),
    )(v_seq2d, nbv, valp, w1a, w1b, w1c, b1, w2, b2)


def kernel(v_seq, T_clip, nb_vecs, W1, b1, W2, b2):
    v2d = v_seq.reshape(R, Cv)
    valp, idx = _sim_topk(v2d, T_clip)

    nb_pad = jnp.concatenate(
        [nb_vecs, jnp.zeros((V, DP - Dnb), jnp.float32)], axis=1)
    rows = _gather_sc(nb_pad, idx.reshape(R * K))
    nbv = rows.reshape(R, K * DP)

    w1a = W1[:Cv]
    w1b = jnp.concatenate(
        [W1[Cv:Cv + K * Dnb].reshape(K, Dnb, H),
         jnp.zeros((K, DP - Dnb, H), jnp.float32)], axis=1).reshape(K * DP, H)
    w1c = jnp.concatenate(
        [W1[Cv + K * Dnb:], jnp.zeros((128 - K, H), jnp.float32)], axis=0)

    out = _mlp(v2d, nbv, valp, w1a, w1b, w1c,
               b1.reshape(1, H), W2, b2.reshape(1, Cv))
    return out.reshape(B, P, Cv)


# speedup vs baseline: 1.4384x; 1.0004x over previous
import functools

import jax
import jax.numpy as jnp
import numpy as np
from jax import lax
from jax.experimental import pallas as pl
from jax.experimental.pallas import tpu as pltpu
from jax.experimental.pallas import tpu_sc as plsc

B, P, Cv = 4, 576, 768
V, Dnb, K, H = 100000, 300, 3, 768
TOTAL_IN = Cv + K * Dnb + K
R = B * P
RB = 576
NRB = R // RB
VB = 2048
NVB = (V + VB - 1) // VB
DP = 384
INT_MAX = np.int32(2**31 - 1)
NEG = np.float32(-np.inf)


def _simtopk_body(vseq_ref, t_ref, vals_ref, idx_ref, qbf_scr, sa_scr,
                  sb_scr, io_scr, rv_scr, ri_scr):
    i = pl.program_id(0)
    j = pl.program_id(1)

    @pl.when(jnp.logical_and(i == 0, j == 0))
    def _once():
        io_scr[...] = lax.broadcasted_iota(jnp.int32, (RB, VB), 1)

    @pl.when(j == 0)
    def _init():
        v = vseq_ref[pl.ds(i * RB, RB), :]
        nrm = jnp.sqrt(jnp.sum(v * v, axis=1, keepdims=True))
        q = v / jnp.maximum(nrm, 1e-12)
        qbf_scr[...] = q.astype(jnp.bfloat16)
        rv_scr[...] = jnp.full((RB, 8), NEG, jnp.float32)
        ri_scr[...] = jnp.zeros((RB, 8), jnp.int32)

    q = qbf_scr[...]
    t = t_ref[...].astype(jnp.bfloat16)

    def _update(s_raw):
        jj = j - 1
        ids = io_scr[...]
        bound = jnp.where(j == 0, 0, V - jj * VB)
        s = jnp.where(ids < bound, s_raw, NEG)

        bv, bi = [], []
        masked = s
        for m in range(K):
            mx = jnp.max(masked, axis=1, keepdims=True)
            am = jnp.min(jnp.where(masked == mx, ids, INT_MAX), axis=1,
                         keepdims=True)
            bv.append(mx)
            bi.append(am + jj * VB)
            if m < K - 1:
                masked = jnp.where(ids == am, NEG, masked)

        cv = jnp.concatenate([rv_scr[...][:, :K]] + bv, axis=1)
        ci = jnp.concatenate([ri_scr[...][:, :K]] + bi, axis=1)
        mv, mi = [], []
        for m in range(K):
            mx = jnp.max(cv, axis=1, keepdims=True)
            am = jnp.min(jnp.where(cv == mx, ci, INT_MAX), axis=1,
                         keepdims=True)
            mv.append(mx)
            mi.append(am)
            if m < K - 1:
                cv = jnp.where(ci == am, NEG, cv)
        rv_scr[...] = jnp.concatenate(
            mv + [jnp.full((RB, 8 - K), NEG, jnp.float32)], axis=1)
        ri_scr[...] = jnp.concatenate(
            mi + [jnp.zeros((RB, 8 - K), jnp.int32)], axis=1)

    dims = (((1,), (1,)), ((), ()))

    @pl.when(j % 2 == 0)
    def _even():
        sa_scr[...] = lax.dot_general(q, t, dims,
                                      preferred_element_type=jnp.float32)
        _update(sb_scr[...])

    @pl.when(j % 2 == 1)
    def _odd():
        sb_scr[...] = lax.dot_general(q, t, dims,
                                      preferred_element_type=jnp.float32)
        _update(sa_scr[...])

    @pl.when(j == NVB)
    def _emit():
        nv = rv_scr[...]
        ni = ri_scr[...]
        vals_ref[pl.ds(i * RB, RB), :] = jnp.concatenate(
            [nv[:, :K], jnp.zeros((RB, 128 - K), jnp.float32)], axis=1)
        idx_ref[pl.ds(i * RB, RB), :] = ni[:, :K]


def _sim_topk(v_seq2d, t_clip):
    grid = (NRB, NVB + 1)
    return pl.pallas_call(
        _simtopk_body,
        grid=grid,
        in_specs=[
            pl.BlockSpec((R, Cv), lambda i, j: (0, 0)),
            pl.BlockSpec((VB, Cv), lambda i, j: (jnp.minimum(j, NVB - 1), 0)),
        ],
        out_specs=[
            pl.BlockSpec((R, 128), lambda i, j: (0, 0)),
            pl.BlockSpec((R, K), lambda i, j: (0, 0)),
        ],
        out_shape=[
            jax.ShapeDtypeStruct((R, 128), jnp.float32),
            jax.ShapeDtypeStruct((R, K), jnp.int32),
        ],
        scratch_shapes=[
            pltpu.VMEM((RB, Cv), jnp.bfloat16),
            pltpu.VMEM((RB, VB), jnp.float32),
            pltpu.VMEM((RB, VB), jnp.float32),
            pltpu.VMEM((RB, VB), jnp.int32),
            pltpu.VMEM((RB, 8), jnp.float32),
            pltpu.VMEM((RB, 8), jnp.int32),
        ],
        compiler_params=pltpu.CompilerParams(
            dimension_semantics=("arbitrary", "arbitrary"),
        ),
    )(v_seq2d, t_clip)


PCH = 160
PSPAN = 3200


def _pad_table_sc(nb_vecs):
    mesh = plsc.VectorSubcoreMesh(core_axis_name="c", subcore_axis_name="s")

    @functools.partial(
        pl.kernel,
        mesh=mesh,
        out_type=jax.ShapeDtypeStruct((V, DP), jnp.float32),
        scratch_types=[
            pltpu.VMEM((PCH, Dnb), jnp.float32),
            pltpu.VMEM((PCH, DP), jnp.float32),
            pltpu.SemaphoreType.DMA,
        ],
    )
    def k(nb_hbm, out_hbm, bufin, buf, sem):
        wid = lax.axis_index("s") * 2 + lax.axis_index("c")
        base = wid * PSPAN
        zeros16 = jnp.zeros((16,), jnp.float32)

        def zrow(r, _):
            for c in (Dnb, Dnb + 16, Dnb + 32, Dnb + 48, Dnb + 64, DP - 16):
                buf[r, pl.ds(c, 16)] = zeros16
            return 0
        lax.fori_loop(0, PCH, zrow, 0)

        cols = tuple(range(0, Dnb - 16, 16)) + (Dnb - 16,)

        def chunk(n, _):
            r0 = base + n * PCH
            pltpu.async_copy(nb_hbm.at[pl.ds(r0, PCH), :], bufin, sem).wait()

            def arow(r, _):
                for c in cols:
                    buf[r, pl.ds(c, 16)] = bufin[r, pl.ds(c, 16)]
                return 0
            lax.fori_loop(0, PCH, arow, 0)
            pltpu.async_copy(buf, out_hbm.at[pl.ds(r0, PCH), :], sem).wait()
            return 0

        nchunks = jnp.where(wid == 31, (V - 31 * PSPAN) // PCH, PSPAN // PCH)
        lax.fori_loop(0, nchunks, chunk, 0)

    return k(nb_vecs)


NW = 32
BPW = (R * K) // NW


def _gather_sc(nb_pad, idx_flat):
    mesh = plsc.VectorSubcoreMesh(core_axis_name="c", subcore_axis_name="s")

    @functools.partial(
        pl.kernel,
        mesh=mesh,
        out_type=jax.ShapeDtypeStruct((R * K, DP), jnp.float32),
        scratch_types=[
            pltpu.VMEM((BPW,), jnp.int32),
            pltpu.VMEM((BPW, DP), jnp.float32),
            pltpu.SemaphoreType.DMA,
        ],
    )
    def k(table_hbm, idx_hbm, out_hbm, idx_v, rows_v, sem):
        wid = lax.axis_index("s") * 2 + lax.axis_index("c")
        base = wid * BPW
        pltpu.sync_copy(idx_hbm.at[pl.ds(base, BPW)], idx_v)
        pltpu.async_copy(table_hbm.at[idx_v], rows_v, sem).wait()
        pltpu.sync_copy(rows_v, out_hbm.at[pl.ds(base, BPW)])

    return k(nb_pad, idx_flat)


MRB = 576


def _mlp_body(vseq_ref, nbv_ref, valp_ref, w1_ref,
              b1_ref, w2_ref, b2_ref, out_ref):
    dims = (((1,), (0,)), ((), ()))
    acc = lax.dot_general(vseq_ref[...].astype(jnp.bfloat16),
                          w1_ref[:Cv].astype(jnp.bfloat16), dims,
                          preferred_element_type=jnp.float32)
    for k in range(K):
        acc += lax.dot_general(
            nbv_ref[:, k * DP:k * DP + Dnb].astype(jnp.bfloat16),
            w1_ref[Cv + k * Dnb:Cv + (k + 1) * Dnb].astype(jnp.bfloat16),
            dims, preferred_element_type=jnp.float32)
    acc += lax.dot_general(valp_ref[:, :K].astype(jnp.bfloat16),
                           w1_ref[Cv + K * Dnb:].astype(jnp.bfloat16), dims,
                           preferred_element_type=jnp.float32)
    acc += b1_ref[...]
    h = 0.5 * acc * (1.0 + lax.erf(acc * np.float32(1.0 / np.sqrt(2.0))))
    f = lax.dot_general(h.astype(jnp.bfloat16),
                        w2_ref[...].astype(jnp.bfloat16), dims,
                        preferred_element_type=jnp.float32)
    f += b2_ref[...]
    mu = jnp.mean(f, axis=1, keepdims=True)
    c = f - mu
    var = jnp.mean(c * c, axis=1, keepdims=True)
    out_ref[...] = c * lax.rsqrt(var + 1e-5)


def _mlp(v_seq2d, nbv, valp, w1, b1, w2, b2):
    grid = (R // MRB,)
    return pl.pallas_call(
        _mlp_body,
        grid=grid,
        in_specs=[
            pl.BlockSpec((MRB, Cv), lambda i: (i, 0)),
            pl.BlockSpec((MRB, K * DP), lambda i: (i, 0)),
            pl.BlockSpec((MRB, 128), lambda i: (i, 0)),
            pl.BlockSpec((TOTAL_IN, H), lambda i: (0, 0)),
            pl.BlockSpec((1, H), lambda i: (0, 0)),
            pl.BlockSpec((H, Cv), lambda i: (0, 0)),
            pl.BlockSpec((1, Cv), lambda i: (0, 0)),
        ],
        out_specs=pl.BlockSpec((MRB, Cv), lambda i: (i, 0)),
        out_shape=jax.ShapeDtypeStruct((R, Cv), jnp.float32),
        compiler_params=pltpu.CompilerParams(
            dimension_semantics=("arbitrary",),
        ),
    )(v_seq2d, nbv, valp, w1, b1, w2, b2)


def kernel(v_seq, T_clip, nb_vecs, W1, b1, W2, b2):
    v2d = v_seq.reshape(R, Cv)
    valp, idx = _sim_topk(v2d, T_clip)

    nb_pad = jnp.concatenate(
        [nb_vecs, jnp.zeros((V, DP - Dnb), jnp.float32)], axis=1)
    rows = _gather_sc(nb_pad, idx.reshape(R * K))
    nbv = rows.reshape(R, K * DP)

    out = _mlp(v2d, nbv, valp, W1,
               b1.reshape(1, H), W2, b2.reshape(1, Cv))
    return out.reshape(B, P, Cv)


# speedup vs baseline: 1.4392x; 1.0006x over previous
import functools

import jax
import jax.numpy as jnp
import numpy as np
from jax import lax
from jax.experimental import pallas as pl
from jax.experimental.pallas import tpu as pltpu
from jax.experimental.pallas import tpu_sc as plsc

B, P, Cv = 4, 576, 768
V, Dnb, K, H = 100000, 300, 3, 768
TOTAL_IN = Cv + K * Dnb + K
R = B * P
RB = 576
NRB = R // RB
VB = 2048
NVB = (V + VB - 1) // VB
DP = 384
INT_MAX = np.int32(2**31 - 1)
NEG = np.float32(-np.inf)


def _simtopk_body(vseq_ref, t_ref, vals_ref, idx_ref, qbf_scr, sa_scr,
                  sb_scr, io_scr, rv_scr, ri_scr):
    i = pl.program_id(0)
    j = pl.program_id(1)

    @pl.when(jnp.logical_and(i == 0, j == 0))
    def _once():
        io_scr[...] = lax.broadcasted_iota(jnp.int32, (RB, VB), 1)

    @pl.when(j == 0)
    def _init():
        v = vseq_ref[pl.ds(i * RB, RB), :]
        nrm = jnp.sqrt(jnp.sum(v * v, axis=1, keepdims=True))
        q = v / jnp.maximum(nrm, 1e-12)
        qbf_scr[...] = q.astype(jnp.bfloat16)
        rv_scr[...] = jnp.full((RB, 8), NEG, jnp.float32)
        ri_scr[...] = jnp.zeros((RB, 8), jnp.int32)

    q = qbf_scr[...]
    t = t_ref[...].astype(jnp.bfloat16)

    def _update(s_raw):
        jj = j - 1
        ids = io_scr[...]
        bound = jnp.where(j == 0, 0, V - jj * VB)
        s = jnp.where(ids < bound, s_raw, NEG)

        bv, bi = [], []
        masked = s
        for m in range(K):
            mx = jnp.max(masked, axis=1, keepdims=True)
            am = jnp.min(jnp.where(masked == mx, ids, INT_MAX), axis=1,
                         keepdims=True)
            bv.append(mx)
            bi.append(am + jj * VB)
            if m < K - 1:
                masked = jnp.where(ids == am, NEG, masked)

        cv = jnp.concatenate([rv_scr[...][:, :K]] + bv, axis=1)
        ci = jnp.concatenate([ri_scr[...][:, :K]] + bi, axis=1)
        mv, mi = [], []
        for m in range(K):
            mx = jnp.max(cv, axis=1, keepdims=True)
            am = jnp.min(jnp.where(cv == mx, ci, INT_MAX), axis=1,
                         keepdims=True)
            mv.append(mx)
            mi.append(am)
            if m < K - 1:
                cv = jnp.where(ci == am, NEG, cv)
        rv_scr[...] = jnp.concatenate(
            mv + [jnp.full((RB, 8 - K), NEG, jnp.float32)], axis=1)
        ri_scr[...] = jnp.concatenate(
            mi + [jnp.zeros((RB, 8 - K), jnp.int32)], axis=1)

    dims = (((1,), (1,)), ((), ()))

    @pl.when(j % 2 == 0)
    def _even():
        sa_scr[...] = lax.dot_general(q, t, dims,
                                      preferred_element_type=jnp.float32)
        _update(sb_scr[...])

    @pl.when(j % 2 == 1)
    def _odd():
        sb_scr[...] = lax.dot_general(q, t, dims,
                                      preferred_element_type=jnp.float32)
        _update(sa_scr[...])

    @pl.when(j == NVB)
    def _emit():
        nv = rv_scr[...]
        ni = ri_scr[...]
        vals_ref[pl.ds(i * RB, RB), :] = jnp.concatenate(
            [nv[:, :K], jnp.zeros((RB, 128 - K), jnp.float32)], axis=1)
        idx_ref[pl.ds(i * RB, RB), :] = ni[:, :K]


def _sim_topk(v_seq2d, t_clip):
    grid = (NRB, NVB + 1)
    return pl.pallas_call(
        _simtopk_body,
        grid=grid,
        in_specs=[
            pl.BlockSpec((R, Cv), lambda i, j: (0, 0)),
            pl.BlockSpec((VB, Cv), lambda i, j: (jnp.minimum(j, NVB - 1), 0)),
        ],
        out_specs=[
            pl.BlockSpec((R, 128), lambda i, j: (0, 0)),
            pl.BlockSpec((R, K), lambda i, j: (0, 0)),
        ],
        out_shape=[
            jax.ShapeDtypeStruct((R, 128), jnp.float32),
            jax.ShapeDtypeStruct((R, K), jnp.int32),
        ],
        scratch_shapes=[
            pltpu.VMEM((RB, Cv), jnp.bfloat16),
            pltpu.VMEM((RB, VB), jnp.float32),
            pltpu.VMEM((RB, VB), jnp.float32),
            pltpu.VMEM((RB, VB), jnp.int32),
            pltpu.VMEM((RB, 8), jnp.float32),
            pltpu.VMEM((RB, 8), jnp.int32),
        ],
        compiler_params=pltpu.CompilerParams(
            dimension_semantics=("arbitrary", "arbitrary"),
        ),
    )(v_seq2d, t_clip)


NW = 32
BPW = (R * K) // NW


def _gather_sc(nb_pad, idx_flat):
    mesh = plsc.VectorSubcoreMesh(core_axis_name="c", subcore_axis_name="s")

    @functools.partial(
        pl.kernel,
        mesh=mesh,
        out_type=jax.ShapeDtypeStruct((R * K, DP), jnp.float32),
        scratch_types=[
            pltpu.VMEM((BPW,), jnp.int32),
            pltpu.VMEM((BPW, DP), jnp.float32),
            pltpu.SemaphoreType.DMA,
        ],
    )
    def k(table_hbm, idx_hbm, out_hbm, idx_v, rows_v, sem):
        wid = lax.axis_index("s") * 2 + lax.axis_index("c")
        base = wid * BPW
        pltpu.sync_copy(idx_hbm.at[pl.ds(base, BPW)], idx_v)
        pltpu.async_copy(table_hbm.at[idx_v], rows_v, sem).wait()
        pltpu.sync_copy(rows_v, out_hbm.at[pl.ds(base, BPW)])

    return k(nb_pad, idx_flat)


MRB = 576


def _mlp_body(vseq_ref, nbv_ref, valp_ref, w1_ref,
              b1_ref, w2_ref, b2_ref, out_ref):
    dims = (((1,), (0,)), ((), ()))
    acc = lax.dot_general(vseq_ref[...].astype(jnp.bfloat16),
                          w1_ref[:Cv].astype(jnp.bfloat16), dims,
                          preferred_element_type=jnp.float32)
    for k in range(K):
        acc += lax.dot_general(
            nbv_ref[:, k * DP:k * DP + Dnb].astype(jnp.bfloat16),
            w1_ref[Cv + k * Dnb:Cv + (k + 1) * Dnb].astype(jnp.bfloat16),
            dims, preferred_element_type=jnp.float32)
    acc += lax.dot_general(valp_ref[:, :K].astype(jnp.bfloat16),
                           w1_ref[Cv + K * Dnb:].astype(jnp.bfloat16), dims,
                           preferred_element_type=jnp.float32)
    acc += b1_ref[...]
    h = 0.5 * acc * (1.0 + lax.erf(acc * np.float32(1.0 / np.sqrt(2.0))))
    f = lax.dot_general(h.astype(jnp.bfloat16),
                        w2_ref[...].astype(jnp.bfloat16), dims,
                        preferred_element_type=jnp.float32)
    f += b2_ref[...]
    mu = jnp.mean(f, axis=1, keepdims=True)
    c = f - mu
    var = jnp.mean(c * c, axis=1, keepdims=True)
    out_ref[...] = c * lax.rsqrt(var + 1e-5)


def _mlp(v_seq2d, nbv, valp, w1, b1, w2, b2):
    grid = (R // MRB,)
    return pl.pallas_call(
        _mlp_body,
        grid=grid,
        in_specs=[
            pl.BlockSpec((MRB, Cv), lambda i: (i, 0)),
            pl.BlockSpec((MRB, K * DP), lambda i: (i, 0)),
            pl.BlockSpec((MRB, 128), lambda i: (i, 0)),
            pl.BlockSpec((TOTAL_IN, H), lambda i: (0, 0)),
            pl.BlockSpec((1, H), lambda i: (0, 0)),
            pl.BlockSpec((H, Cv), lambda i: (0, 0)),
            pl.BlockSpec((1, Cv), lambda i: (0, 0)),
        ],
        out_specs=pl.BlockSpec((MRB, Cv), lambda i: (i, 0)),
        out_shape=jax.ShapeDtypeStruct((R, Cv), jnp.float32),
        compiler_params=pltpu.CompilerParams(
            dimension_semantics=("arbitrary",),
        ),
    )(v_seq2d, nbv, valp, w1, b1, w2, b2)


def kernel(v_seq, T_clip, nb_vecs, W1, b1, W2, b2):
    v2d = v_seq.reshape(R, Cv)
    valp, idx = _sim_topk(v2d, T_clip)

    nb_pad = jnp.concatenate(
        [nb_vecs, jnp.zeros((V, DP - Dnb), jnp.float32)], axis=1)
    rows = _gather_sc(nb_pad, idx.reshape(R * K))
    nbv = rows.reshape(R, K * DP)

    out = _mlp(v2d, nbv, valp, W1,
               b1.reshape(1, H), W2, b2.reshape(1, Cv))
    return out.reshape(B, P, Cv)
